# baseline (speedup 1.0000x reference)
import functools

import jax
import jax.numpy as jnp
from jax.experimental import pallas as pl
from jax.experimental.pallas import tpu as pltpu

_INV_SQRT2 = 0.7071067811865476


def _router_kernel(tok_ref, gamma_ref, beta_ref, w1_ref, b1_ref, w2_ref,
                   b2_ref, logits_ref, se_ref, ew_ref, xn_ref, acc_ref,
                   *, nh, e):
    h_idx = pl.program_id(1)

    @pl.when(h_idx == 0)
    def _layernorm():
        x = tok_ref[...]
        mu = jnp.mean(x, axis=-1, keepdims=True)
        xc = x - mu
        var = jnp.mean(xc * xc, axis=-1, keepdims=True)
        xn = xc * jax.lax.rsqrt(var + 1e-5) * gamma_ref[...] + beta_ref[...]
        xn_ref[...] = xn.astype(jnp.bfloat16)

    xn = xn_ref[...]
    bh = w1_ref.shape[1]
    ck = min(256, bh)
    part = None
    for j in range(0, bh, ck):
        hblk = jnp.dot(xn, w1_ref[:, j:j + ck],
                       preferred_element_type=jnp.float32)
        hblk = hblk + b1_ref[:, j:j + ck]
        g = hblk * 0.5 * (1.0 + jax.lax.erf(hblk * _INV_SQRT2))
        pj = jnp.dot(g.astype(jnp.bfloat16), w2_ref[j:j + ck, :],
                     preferred_element_type=jnp.float32)
        part = pj if part is None else part + pj

    @pl.when(h_idx == 0)
    def _init():
        acc_ref[...] = part

    @pl.when(h_idx > 0)
    def _accum():
        acc_ref[...] += part

    @pl.when(h_idx == nh - 1)
    def _finalize():
        logits = acc_ref[...] + b2_ref[...]
        logits_ref[...] = logits
        col = jax.lax.broadcasted_iota(jnp.int32, logits.shape, 1)
        m1 = jnp.max(logits, axis=1, keepdims=True)
        i1 = jnp.min(jnp.where(logits == m1, col, e), axis=1, keepdims=True)
        masked = jnp.where(col == i1, -jnp.inf, logits)
        m2 = jnp.max(masked, axis=1, keepdims=True)
        i2 = jnp.min(jnp.where(masked == m2, col, e), axis=1, keepdims=True)
        t = jnp.exp(m2 - m1)
        s = 1.0 + t
        wa = 1.0 / s
        wb = t / s
        ew_ref[...] = jnp.where(col == i1, wa,
                                jnp.where(col == i2, wb, 0.0))
        se_ref[...] = jnp.concatenate([i1, i2], axis=1)


def kernel(tokens, gamma, beta, W1, b1, W2, b2):
    n, d = tokens.shape
    h = W1.shape[1]
    e = W2.shape[1]
    bm = min(512, n)
    bh = min(1024, h)
    grid = (n // bm, h // bh)

    out = pl.pallas_call(
        functools.partial(_router_kernel, nh=grid[1], e=e),
        grid=grid,
        in_specs=[
            pl.BlockSpec((bm, d), lambda m, hh: (m, 0)),
            pl.BlockSpec((1, d), lambda m, hh: (0, 0)),
            pl.BlockSpec((1, d), lambda m, hh: (0, 0)),
            pl.BlockSpec((d, bh), lambda m, hh: (0, hh)),
            pl.BlockSpec((1, bh), lambda m, hh: (0, hh)),
            pl.BlockSpec((bh, e), lambda m, hh: (hh, 0)),
            pl.BlockSpec((1, e), lambda m, hh: (0, 0)),
        ],
        out_specs=[
            pl.BlockSpec((bm, e), lambda m, hh: (m, 0)),
            pl.BlockSpec((bm, 2), lambda m, hh: (m, 0)),
            pl.BlockSpec((bm, e), lambda m, hh: (m, 0)),
        ],
        out_shape=[
            jax.ShapeDtypeStruct((n, e), jnp.float32),
            jax.ShapeDtypeStruct((n, 2), jnp.int32),
            jax.ShapeDtypeStruct((n, e), jnp.float32),
        ],
        scratch_shapes=[
            pltpu.VMEM((bm, d), jnp.bfloat16),
            pltpu.VMEM((bm, e), jnp.float32),
        ],
    )(tokens, gamma.reshape(1, d), beta.reshape(1, d),
      W1.astype(jnp.bfloat16), b1.reshape(1, h),
      W2.astype(jnp.bfloat16), b2.reshape(1, e))
    return (out[0], out[1], out[2])


# speedup vs baseline: 1.3374x; 1.3374x over previous
import functools

import jax
import jax.numpy as jnp
from jax.experimental import pallas as pl
from jax.experimental.pallas import tpu as pltpu

_INV_SQRT2 = 0.7071067811865476


def _gelu_dot2(hblk, w2):
    g = hblk * 0.5 * (1.0 + jax.lax.erf(hblk * _INV_SQRT2))
    return jnp.dot(g.astype(jnp.bfloat16), w2,
                   preferred_element_type=jnp.float32)


def _finalize(logits, logits_ref, se_ref, ew_ref, e):
    logits_ref[...] = logits
    col = jax.lax.broadcasted_iota(jnp.int32, logits.shape, 1)
    m1 = jnp.max(logits, axis=1, keepdims=True)
    i1 = jnp.min(jnp.where(logits == m1, col, e), axis=1, keepdims=True)
    masked = jnp.where(col == i1, -jnp.inf, logits)
    m2 = jnp.max(masked, axis=1, keepdims=True)
    i2 = jnp.min(jnp.where(masked == m2, col, e), axis=1, keepdims=True)
    t = jnp.exp(m2 - m1)
    s = 1.0 + t
    wa = 1.0 / s
    wb = t / s
    ew_ref[...] = jnp.where(col == i1, wa, jnp.where(col == i2, wb, 0.0))
    se_ref[...] = jnp.concatenate([i1, i2], axis=1)


def _router_kernel(tok_ref, gamma_ref, beta_ref, w1_ref, b1_ref, w2_ref,
                   b2_ref, logits_ref, se_ref, ew_ref, xn_ref, hbuf_ref,
                   acc_ref, *, nh, e):
    h_idx = pl.program_id(1)

    @pl.when(h_idx == 0)
    def _layernorm():
        x = tok_ref[...]
        mu = jnp.mean(x, axis=-1, keepdims=True)
        xc = x - mu
        var = jnp.mean(xc * xc, axis=-1, keepdims=True)
        xn = xc * jax.lax.rsqrt(var + 1e-5) * gamma_ref[...] + beta_ref[...]
        xn_ref[...] = xn.astype(jnp.bfloat16)

    prev = (h_idx - 1) & 1
    hprev = hbuf_ref[prev]
    pj = _gelu_dot2(hprev, w2_ref[jnp.maximum(h_idx - 1, 0)])
    acc_ref[...] = jnp.where(h_idx <= 1, 0.0, acc_ref[...]) + pj

    hblk = jnp.dot(xn_ref[...], w1_ref[...],
                   preferred_element_type=jnp.float32) + b1_ref[0]
    hbuf_ref[h_idx & 1] = hblk

    @pl.when(h_idx == nh - 1)
    def _drain_and_route():
        plast = _gelu_dot2(hblk, w2_ref[nh - 1])
        logits = acc_ref[...] + plast + b2_ref[...]
        _finalize(logits, logits_ref, se_ref, ew_ref, e)


def kernel(tokens, gamma, beta, W1, b1, W2, b2):
    n, d = tokens.shape
    h = W1.shape[1]
    e = W2.shape[1]
    bm = min(512, n)
    bh = min(1024, h)
    nh = h // bh
    grid = (n // bm, nh)

    out = pl.pallas_call(
        functools.partial(_router_kernel, nh=nh, e=e),
        grid=grid,
        in_specs=[
            pl.BlockSpec((bm, d), lambda m, hh: (m, 0)),
            pl.BlockSpec((1, d), lambda m, hh: (0, 0)),
            pl.BlockSpec((1, d), lambda m, hh: (0, 0)),
            pl.BlockSpec((d, bh), lambda m, hh: (0, hh)),
            pl.BlockSpec((1, 1, bh), lambda m, hh: (hh, 0, 0)),
            pl.BlockSpec((nh, bh, e), lambda m, hh: (0, 0, 0)),
            pl.BlockSpec((1, e), lambda m, hh: (0, 0)),
        ],
        out_specs=[
            pl.BlockSpec((bm, e), lambda m, hh: (m, 0)),
            pl.BlockSpec((bm, 2), lambda m, hh: (m, 0)),
            pl.BlockSpec((bm, e), lambda m, hh: (m, 0)),
        ],
        out_shape=[
            jax.ShapeDtypeStruct((n, e), jnp.float32),
            jax.ShapeDtypeStruct((n, 2), jnp.int32),
            jax.ShapeDtypeStruct((n, e), jnp.float32),
        ],
        scratch_shapes=[
            pltpu.VMEM((bm, d), jnp.bfloat16),
            pltpu.VMEM((2, bm, bh), jnp.float32),
            pltpu.VMEM((bm, e), jnp.float32),
        ],
    )(tokens, gamma.reshape(1, d), beta.reshape(1, d),
      W1.astype(jnp.bfloat16), b1.reshape(nh, 1, bh),
      W2.astype(jnp.bfloat16).reshape(nh, bh, e), b2.reshape(1, e))
    return (out[0], out[1], out[2])


# speedup vs baseline: 1.3386x; 1.0009x over previous
import functools

import jax
import jax.numpy as jnp
from jax.experimental import pallas as pl
from jax.experimental.pallas import tpu as pltpu

_INV_SQRT2 = 0.7071067811865476


def _gelu_dot2(hblk, w2):
    g = hblk * 0.5 * (1.0 + jax.lax.erf(hblk * _INV_SQRT2))
    return jnp.dot(g.astype(jnp.bfloat16), w2,
                   preferred_element_type=jnp.float32)


def _finalize(logits, logits_ref, se_ref, ew_ref, e):
    logits_ref[...] = logits
    col = jax.lax.broadcasted_iota(jnp.int32, logits.shape, 1)
    m1 = jnp.max(logits, axis=1, keepdims=True)
    i1 = jnp.min(jnp.where(logits == m1, col, e), axis=1, keepdims=True)
    masked = jnp.where(col == i1, -jnp.inf, logits)
    m2 = jnp.max(masked, axis=1, keepdims=True)
    i2 = jnp.min(jnp.where(masked == m2, col, e), axis=1, keepdims=True)
    t = jnp.exp(m2 - m1)
    s = 1.0 + t
    wa = 1.0 / s
    wb = t / s
    ew_ref[...] = jnp.where(col == i1, wa, jnp.where(col == i2, wb, 0.0))
    se_ref[...] = jnp.concatenate([i1, i2], axis=1)


def _router_kernel(tok_ref, gamma_ref, beta_ref, w1a_ref, w1b_ref,
                   b1a_ref, b1b_ref, w2p0_ref, w2p1_ref, w2last_ref,
                   b2_ref, logits_ref, se_ref, ew_ref,
                   xn_ref, hprev_ref, acc_ref, *, nt, e):
    t_idx = pl.program_id(1)

    @pl.when(t_idx == 0)
    def _layernorm():
        x = tok_ref[...]
        mu = jnp.mean(x, axis=-1, keepdims=True)
        xc = x - mu
        var = jnp.mean(xc * xc, axis=-1, keepdims=True)
        xn = xc * jax.lax.rsqrt(var + 1e-5) * gamma_ref[...] + beta_ref[...]
        xn_ref[...] = xn.astype(jnp.bfloat16)

    xn = xn_ref[...]
    p0 = _gelu_dot2(hprev_ref[...], w2p0_ref[...])
    hA = jnp.dot(xn, w1a_ref[...],
                 preferred_element_type=jnp.float32) + b1a_ref[...]
    p1 = _gelu_dot2(hA, w2p1_ref[...])
    hB = jnp.dot(xn, w1b_ref[...],
                 preferred_element_type=jnp.float32) + b1b_ref[...]
    hprev_ref[...] = hB
    acc_ref[...] = jnp.where(t_idx == 0, 0.0, acc_ref[...] + p0) + p1

    @pl.when(t_idx == nt - 1)
    def _drain_and_route():
        plast = _gelu_dot2(hB, w2last_ref[...])
        logits = acc_ref[...] + plast + b2_ref[...]
        _finalize(logits, logits_ref, se_ref, ew_ref, e)


def kernel(tokens, gamma, beta, W1, b1, W2, b2):
    n, d = tokens.shape
    h = W1.shape[1]
    e = W2.shape[1]
    bm = min(512, n)
    bh = min(512, h)
    nh = h // bh
    nt = nh // 2
    grid = (n // bm, nt)

    w1r = W1.astype(jnp.bfloat16)
    w2r = W2.astype(jnp.bfloat16)
    b1r = b1.reshape(1, h)

    out = pl.pallas_call(
        functools.partial(_router_kernel, nt=nt, e=e),
        grid=grid,
        in_specs=[
            pl.BlockSpec((bm, d), lambda m, t: (m, 0)),
            pl.BlockSpec((1, d), lambda m, t: (0, 0)),
            pl.BlockSpec((1, d), lambda m, t: (0, 0)),
            pl.BlockSpec((d, bh), lambda m, t: (0, 2 * t)),
            pl.BlockSpec((d, bh), lambda m, t: (0, 2 * t + 1)),
            pl.BlockSpec((1, bh), lambda m, t: (0, 2 * t)),
            pl.BlockSpec((1, bh), lambda m, t: (0, 2 * t + 1)),
            pl.BlockSpec((bh, e),
                         lambda m, t: (jnp.maximum(2 * t - 1, 0), 0)),
            pl.BlockSpec((bh, e), lambda m, t: (2 * t, 0)),
            pl.BlockSpec((bh, e), lambda m, t: (2 * t + 1, 0)),
            pl.BlockSpec((1, e), lambda m, t: (0, 0)),
        ],
        out_specs=[
            pl.BlockSpec((bm, e), lambda m, t: (m, 0)),
            pl.BlockSpec((bm, 2), lambda m, t: (m, 0)),
            pl.BlockSpec((bm, e), lambda m, t: (m, 0)),
        ],
        out_shape=[
            jax.ShapeDtypeStruct((n, e), jnp.float32),
            jax.ShapeDtypeStruct((n, 2), jnp.int32),
            jax.ShapeDtypeStruct((n, e), jnp.float32),
        ],
        scratch_shapes=[
            pltpu.VMEM((bm, d), jnp.bfloat16),
            pltpu.VMEM((bm, bh), jnp.float32),
            pltpu.VMEM((bm, e), jnp.float32),
        ],
    )(tokens, gamma.reshape(1, d), beta.reshape(1, d),
      w1r, w1r, b1r, b1r, w2r, w2r, w2r, b2.reshape(1, e))
    return (out[0], out[1], out[2])


# speedup vs baseline: 1.4550x; 1.0870x over previous
import functools

import jax
import jax.numpy as jnp
from jax.experimental import pallas as pl
from jax.experimental.pallas import tpu as pltpu

_INV_SQRT2 = 0.7071067811865476


def _gelu_dot2(hblk, w2):
    g = hblk * 0.5 * (1.0 + jax.lax.erf(hblk * _INV_SQRT2))
    return jnp.dot(g.astype(jnp.bfloat16), w2,
                   preferred_element_type=jnp.float32)


def _router_kernel(*refs, nh, e):
    tok_ref, gamma_ref, beta_ref = refs[0], refs[1], refs[2]
    w1_refs = refs[3:3 + nh]
    b1_refs = refs[3 + nh:3 + 2 * nh]
    w2_refs = refs[3 + 2 * nh:3 + 3 * nh]
    b2_ref = refs[3 + 3 * nh]
    logits_ref, se_ref, ew_ref = refs[3 + 3 * nh + 1:3 + 3 * nh + 4]

    x = tok_ref[...]
    mu = jnp.mean(x, axis=-1, keepdims=True)
    xc = x - mu
    var = jnp.mean(xc * xc, axis=-1, keepdims=True)
    xn32 = xc * jax.lax.rsqrt(var + 1e-5) * gamma_ref[...] + beta_ref[...]
    xn = xn32.astype(jnp.bfloat16)

    acc = None
    hprev = None
    for j in range(nh):
        hj = jnp.dot(xn, w1_refs[j][...],
                     preferred_element_type=jnp.float32) + b1_refs[j][...]
        if hprev is not None:
            pj = _gelu_dot2(hprev, w2_refs[j - 1][...])
            acc = pj if acc is None else acc + pj
        hprev = hj
    plast = _gelu_dot2(hprev, w2_refs[nh - 1][...])
    logits = plast + b2_ref[...] if acc is None else acc + plast + b2_ref[...]

    logits_ref[...] = logits
    col = jax.lax.broadcasted_iota(jnp.int32, logits.shape, 1)
    m1 = jnp.max(logits, axis=1, keepdims=True)
    i1 = jnp.min(jnp.where(logits == m1, col, e), axis=1, keepdims=True)
    masked = jnp.where(col == i1, -jnp.inf, logits)
    m2 = jnp.max(masked, axis=1, keepdims=True)
    i2 = jnp.min(jnp.where(masked == m2, col, e), axis=1, keepdims=True)
    t = jnp.exp(m2 - m1)
    s = 1.0 + t
    wa = 1.0 / s
    wb = t / s
    ew_ref[...] = jnp.where(col == i1, wa, jnp.where(col == i2, wb, 0.0))
    se_ref[...] = jnp.concatenate([i1, i2], axis=1)


def kernel(tokens, gamma, beta, W1, b1, W2, b2):
    n, d = tokens.shape
    h = W1.shape[1]
    e = W2.shape[1]
    bm = min(512, n)
    bh = min(512, h)
    nh = h // bh
    grid = (n // bm,)

    def _const2(i, j):
        return lambda m: (i, j)

    in_specs = [
        pl.BlockSpec((bm, d), lambda m: (m, 0)),
        pl.BlockSpec((1, d), lambda m: (0, 0)),
        pl.BlockSpec((1, d), lambda m: (0, 0)),
    ]
    in_specs += [pl.BlockSpec((d, bh), _const2(0, j)) for j in range(nh)]
    in_specs += [pl.BlockSpec((1, bh), _const2(0, j)) for j in range(nh)]
    in_specs += [pl.BlockSpec((bh, e), _const2(j, 0)) for j in range(nh)]
    in_specs += [pl.BlockSpec((1, e), lambda m: (0, 0))]

    w1b = W1.astype(jnp.bfloat16)
    w2b = W2.astype(jnp.bfloat16)
    b1r = b1.reshape(1, h)

    out = pl.pallas_call(
        functools.partial(_router_kernel, nh=nh, e=e),
        grid=grid,
        compiler_params=pltpu.CompilerParams(
            vmem_limit_bytes=63 * 1024 * 1024),
        in_specs=in_specs,
        out_specs=[
            pl.BlockSpec((bm, e), lambda m: (m, 0)),
            pl.BlockSpec((bm, 2), lambda m: (m, 0)),
            pl.BlockSpec((bm, e), lambda m: (m, 0)),
        ],
        out_shape=[
            jax.ShapeDtypeStruct((n, e), jnp.float32),
            jax.ShapeDtypeStruct((n, 2), jnp.int32),
            jax.ShapeDtypeStruct((n, e), jnp.float32),
        ],
    )(tokens, gamma.reshape(1, d), beta.reshape(1, d),
      *[w1b] * nh, *[b1r] * nh, *[w2b] * nh, b2.reshape(1, e))
    return (out[0], out[1], out[2])
